# baseline (speedup 1.0000x reference)
import functools

import jax
import jax.numpy as jnp
import numpy as np
from jax import lax
from jax.experimental import pallas as pl
from jax.experimental.pallas import tpu as pltpu
from jax.experimental.pallas import tpu_sc as plsc

N = 10000
E = 320000
D = 128
L = 3
K = 32

NC = 2
NS = 16
NW = NC * NS

CH = 128
CPW = 8 * -(-E // (CH * NW * 8))
E_PAD = CPW * CH * NW
N_PAD = 10112
RPS = N_PAD // NS

BR = 512
GRID = -(-N // BR)

_SC_MESH = dict(
    mesh=plsc.VectorSubcoreMesh(core_axis_name="c", subcore_axis_name="s"),
)
_SC_MESH1 = dict(
    mesh=plsc.VectorSubcoreMesh(core_axis_name="c", subcore_axis_name="s",
                                num_cores=1),
)
HROWS = NW * CPW // 2


@functools.partial(
    pl.kernel,
    out_type=jax.ShapeDtypeStruct((NC * N_PAD, D), jnp.float32),
    scratch_types=[
        pltpu.VMEM((CPW, CH), jnp.int32),
        pltpu.VMEM((CH, D), jnp.float32),
        pltpu.VMEM_SHARED((N_PAD, D), jnp.float32),
    ],
    **_SC_MESH,
)
def _sc_deg(dst_hbm, zeros_hbm, ones_hbm, out_hbm, dst_v, ones_v, acc_sh):
    c = lax.axis_index("c")
    s = lax.axis_index("s")
    wid = c * NS + s
    r0 = s * RPS
    pltpu.sync_copy(zeros_hbm.at[pl.ds(r0, RPS)], acc_sh.at[pl.ds(r0, RPS)])
    pltpu.sync_copy(dst_hbm.at[pl.ds(wid * CPW, CPW)], dst_v)
    pltpu.sync_copy(ones_hbm, ones_v)
    plsc.subcore_barrier()

    def body(j, carry):
        pltpu.sync_copy(ones_v, acc_sh.at[dst_v.at[j]], add=True)
        return carry

    lax.fori_loop(0, CPW, body, 0)
    plsc.subcore_barrier()
    pltpu.sync_copy(acc_sh.at[pl.ds(r0, RPS)],
                    out_hbm.at[pl.ds(c * N_PAD + r0, RPS)])


@functools.partial(
    pl.kernel,
    out_type=jax.ShapeDtypeStruct((N_PAD, D), jnp.float32),
    scratch_types=[
        pltpu.VMEM((CPW // 2, CH), jnp.int32),
        pltpu.VMEM((CPW // 2, CH), jnp.int32),
        pltpu.VMEM((CH, D), jnp.float32),
        pltpu.VMEM((CH, D), jnp.float32),
        pltpu.VMEM_SHARED((N_PAD, D), jnp.float32),
        pltpu.SemaphoreType.DMA,
        pltpu.SemaphoreType.DMA,
        pltpu.SemaphoreType.DMA,
        pltpu.SemaphoreType.DMA,
    ],
    **_SC_MESH1,
)
def _sc_spmm(hp_hbm, src_hbm, dst_hbm, zeros_hbm, out_hbm,
             src_v, dst_v, rows0, rows1, acc_sh, gs0, gs1, ss0, ss1):
    s = lax.axis_index("s")
    wid = s
    r0 = s * RPS
    HP = CPW // 2
    pltpu.sync_copy(zeros_hbm.at[pl.ds(r0, RPS)], acc_sh.at[pl.ds(r0, RPS)])
    plsc.subcore_barrier()

    bufs = (rows0, rows1)
    gsems = (gs0, gs1)
    ssems = (ss0, ss1)

    for p in range(2):
        pltpu.sync_copy(src_hbm.at[pl.ds(wid * CPW + p * HP, HP)], src_v)
        pltpu.sync_copy(dst_hbm.at[pl.ds(wid * CPW + p * HP, HP)], dst_v)
        pltpu.async_copy(hp_hbm.at[src_v.at[0]], rows0, gs0)
        pltpu.async_copy(hp_hbm.at[src_v.at[1]], rows1, gs1)

        def body(jj, carry):
            for b in range(2):
                j = jj * 2 + b
                pltpu.make_async_copy(hp_hbm.at[src_v.at[j]], bufs[b],
                                      gsems[b]).wait()
                pltpu.async_copy(bufs[b], acc_sh.at[dst_v.at[j]], ssems[b],
                                 add=True)
                nxt = j + 2

                @pl.when(nxt < HP)
                def _():
                    pltpu.make_async_copy(bufs[b], acc_sh.at[dst_v.at[j]],
                                          ssems[b]).wait()
                    pltpu.async_copy(hp_hbm.at[src_v.at[nxt]], bufs[b],
                                     gsems[b])

            return carry

        lax.fori_loop(0, HP // 2, body, 0)
        for b in range(2):
            pltpu.make_async_copy(bufs[b], acc_sh.at[dst_v.at[HP - 2 + b]],
                                  ssems[b]).wait()
    plsc.subcore_barrier()
    pltpu.sync_copy(acc_sh.at[pl.ds(r0, RPS)], out_hbm.at[pl.ds(r0, RPS)])


def _dotT(x, w):
    return lax.dot_general(x, w, (((1,), (1,)), ((), ())),
                           preferred_element_type=jnp.float32)


_BITS = [int(np.uint32(1 << b).astype(np.int64)) - (1 << 32 if b == 31 else 0)
         for b in range(32)]


def _maxk(y):
    i = lax.bitcast_convert_type(y, jnp.int32)
    ikey = jnp.where(i < 0, i ^ jnp.int32(0x7FFFFFFF), i)
    minint = jnp.int32(-(1 << 31))
    tu = jnp.zeros(y.shape[:-1] + (1,), jnp.int32)
    for b in range(31, -1, -1):
        cand = tu | jnp.int32(_BITS[b])
        cnt = jnp.sum((ikey >= (cand ^ minint)).astype(jnp.int32),
                      axis=-1, keepdims=True)
        tu = jnp.where(cnt >= K, cand, tu)
    thr = tu ^ minint
    return jnp.where(ikey >= thr, y, jnp.zeros_like(y))


def _scale_from_deg(d0, d1):
    deg = jnp.maximum(d0[:, :1] + d1[:, :1], 1.0)
    return lax.rsqrt(deg)


def _tc_in_body(x_ref, win_ref, bin_ref, w0_ref, b0_ref, d0_ref, d1_ref, o_ref):
    h = jnp.maximum(_dotT(x_ref[...], win_ref[...]) + bin_ref[...], 0.0)
    y = _dotT(h, w0_ref[...]) + b0_ref[...]
    o_ref[...] = _maxk(y) * _scale_from_deg(d0_ref[...], d1_ref[...])


def _tc_mid_body(s0_ref, s1_ref, d0_ref, d1_ref, cw_ref, cb_ref,
                 lw_ref, lb_ref, o_ref):
    a = _scale_from_deg(d0_ref[...], d1_ref[...])
    h = _dotT((s0_ref[...] + s1_ref[...]) * a, cw_ref[...]) + cb_ref[...]
    y = _dotT(h, lw_ref[...]) + lb_ref[...]
    o_ref[...] = _maxk(y) * a


def _tc_out_body(s0_ref, s1_ref, d0_ref, d1_ref, cw_ref, cb_ref,
                 wo_ref, bo_ref, o_ref):
    a = _scale_from_deg(d0_ref[...], d1_ref[...])
    h = _dotT((s0_ref[...] + s1_ref[...]) * a, cw_ref[...]) + cb_ref[...]
    o_ref[...] = _dotT(h, wo_ref[...]) + bo_ref[...]


def _row_spec(cols):
    return pl.BlockSpec((BR, cols), lambda i: (i, 0))


def _full_spec(rows, cols):
    return pl.BlockSpec((rows, cols), lambda i: (0, 0))


def _tc_call(body, in_specs, out_rows):
    return pl.pallas_call(
        body,
        grid=(GRID,),
        in_specs=in_specs,
        out_specs=_row_spec(D),
        out_shape=jax.ShapeDtypeStruct((out_rows, D), jnp.float32),
    )


def kernel(x, edge_index, W_in, b_in, lin_W, lin_b, conv_W, conv_b,
           W_out, b_out):
    src = edge_index[0].astype(jnp.int32)
    dst = edge_index[1].astype(jnp.int32)
    pad = E_PAD - E
    src_p = jnp.concatenate([src, jnp.zeros((pad,), jnp.int32)])
    dst_p = jnp.concatenate([dst, jnp.full((pad,), N_PAD - 1, jnp.int32)])
    src2d = src_p.reshape(NW * CPW, CH)
    dst2d = dst_p.reshape(NW * CPW, CH)

    onesD = jnp.ones((CH, D), jnp.float32)
    zerosD = jnp.zeros((N_PAD, D), jnp.float32)

    degp = _sc_deg(dst2d, zerosD, onesD)
    d0 = degp[:N_PAD]
    d1 = degp[N_PAD:]

    deg_specs = [_row_spec(D), _row_spec(D)]
    w_spec = _full_spec(D, D)
    b_spec = _full_spec(1, D)

    hp = _tc_call(
        _tc_in_body,
        [_row_spec(D), w_spec, b_spec, w_spec, b_spec] + deg_specs,
        N,
    )(x, W_in, b_in.reshape(1, D), lin_W[0], lin_b[0].reshape(1, D), d0, d1)

    srcA, srcB = src2d[:HROWS], src2d[HROWS:]
    dstA, dstB = dst2d[:HROWS], dst2d[HROWS:]
    for i in range(L):
        s0 = _sc_spmm(hp, srcA, dstA, zerosD)
        s1 = _sc_spmm(hp, srcB, dstB, zerosD)
        if i < L - 1:
            hp = _tc_call(
                _tc_mid_body,
                [_row_spec(D), _row_spec(D)] + deg_specs
                + [w_spec, b_spec, w_spec, b_spec],
                N,
            )(s0, s1, d0, d1, conv_W[i], conv_b[i].reshape(1, D),
              lin_W[i + 1], lin_b[i + 1].reshape(1, D))
        else:
            out = _tc_call(
                _tc_out_body,
                [_row_spec(D), _row_spec(D)] + deg_specs
                + [w_spec, b_spec, w_spec, b_spec],
                N,
            )(s0, s1, d0, d1, conv_W[i], conv_b[i].reshape(1, D),
              W_out, b_out.reshape(1, D))
    return out


# speedup vs baseline: 1.2129x; 1.2129x over previous
import functools

import jax
import jax.numpy as jnp
import numpy as np
from jax import lax
from jax.experimental import pallas as pl
from jax.experimental.pallas import tpu as pltpu
from jax.experimental.pallas import tpu_sc as plsc

N = 10000
E = 320000
D = 128
L = 3
K = 32

NC = 2
NS = 16
NW = NC * NS

CH = 128
CPW = 8 * -(-E // (CH * NW * 8))
E_PAD = CPW * CH * NW
N_PAD = 10112
RPS = N_PAD // NS

BR = 512
GRID = -(-N // BR)

_SC_MESH = dict(
    mesh=plsc.VectorSubcoreMesh(core_axis_name="c", subcore_axis_name="s"),
)
_SC_MESH1 = dict(
    mesh=plsc.VectorSubcoreMesh(core_axis_name="c", subcore_axis_name="s",
                                num_cores=1),
)
HROWS = NW * CPW // 2


@functools.partial(
    pl.kernel,
    out_type=jax.ShapeDtypeStruct((NC * N_PAD, D), jnp.float32),
    scratch_types=[
        pltpu.VMEM((CPW, CH), jnp.int32),
        pltpu.VMEM((CH, D), jnp.float32),
        pltpu.VMEM_SHARED((N_PAD, D), jnp.float32),
    ],
    **_SC_MESH,
)
def _sc_deg(dst_hbm, zeros_hbm, ones_hbm, out_hbm, dst_v, ones_v, acc_sh):
    c = lax.axis_index("c")
    s = lax.axis_index("s")
    wid = c * NS + s
    r0 = s * RPS
    pltpu.sync_copy(zeros_hbm.at[pl.ds(r0, RPS)], acc_sh.at[pl.ds(r0, RPS)])
    pltpu.sync_copy(dst_hbm.at[pl.ds(wid * CPW, CPW)], dst_v)
    pltpu.sync_copy(ones_hbm, ones_v)
    plsc.subcore_barrier()

    def body(j, carry):
        pltpu.sync_copy(ones_v, acc_sh.at[dst_v.at[j]], add=True)
        return carry

    lax.fori_loop(0, CPW, body, 0)
    plsc.subcore_barrier()
    pltpu.sync_copy(acc_sh.at[pl.ds(r0, RPS)],
                    out_hbm.at[pl.ds(c * N_PAD + r0, RPS)])


@functools.partial(
    pl.kernel,
    out_type=jax.ShapeDtypeStruct((NC * N_PAD, D), jnp.float32),
    scratch_types=[
        pltpu.VMEM((CPW // 2, CH), jnp.int32),
        pltpu.VMEM((CPW // 2, CH), jnp.int32),
        pltpu.VMEM((CH, D), jnp.float32),
        pltpu.VMEM((CH, D), jnp.float32),
        pltpu.VMEM_SHARED((N_PAD, D), jnp.float32),
        pltpu.SemaphoreType.DMA,
        pltpu.SemaphoreType.DMA,
        pltpu.SemaphoreType.DMA,
        pltpu.SemaphoreType.DMA,
    ],
    **_SC_MESH,
)
def _sc_spmm(hp_hbm, src_hbm, dst_hbm, zeros_hbm, out_hbm,
             src_v, dst_v, rows0, rows1, acc_sh, gs0, gs1, ss0, ss1):
    c = lax.axis_index("c")
    s = lax.axis_index("s")
    wid = c * NS + s
    r0 = s * RPS
    HP = CPW // 2
    pltpu.sync_copy(zeros_hbm.at[pl.ds(r0, RPS)], acc_sh.at[pl.ds(r0, RPS)])
    plsc.subcore_barrier()

    bufs = (rows0, rows1)
    gsems = (gs0, gs1)
    ssems = (ss0, ss1)

    for p in range(2):
        pltpu.sync_copy(src_hbm.at[pl.ds(wid * CPW + p * HP, HP)], src_v)
        pltpu.sync_copy(dst_hbm.at[pl.ds(wid * CPW + p * HP, HP)], dst_v)
        pltpu.async_copy(hp_hbm.at[src_v.at[0]], rows0, gs0)
        pltpu.async_copy(hp_hbm.at[src_v.at[1]], rows1, gs1)

        def body(jj, carry):
            for b in range(2):
                j = jj * 2 + b
                pltpu.make_async_copy(hp_hbm.at[src_v.at[j]], bufs[b],
                                      gsems[b]).wait()
                pltpu.async_copy(bufs[b], acc_sh.at[dst_v.at[j]], ssems[b],
                                 add=True)
                nxt = j + 2

                @pl.when(nxt < HP)
                def _():
                    pltpu.make_async_copy(bufs[b], acc_sh.at[dst_v.at[j]],
                                          ssems[b]).wait()
                    pltpu.async_copy(hp_hbm.at[src_v.at[nxt]], bufs[b],
                                     gsems[b])

            return carry

        lax.fori_loop(0, HP // 2, body, 0)
        for b in range(2):
            pltpu.make_async_copy(bufs[b], acc_sh.at[dst_v.at[HP - 2 + b]],
                                  ssems[b]).wait()
    plsc.subcore_barrier()
    pltpu.sync_copy(acc_sh.at[pl.ds(r0, RPS)],
                    out_hbm.at[pl.ds(c * N_PAD + r0, RPS)])


def _dotT(x, w):
    return lax.dot_general(x, w, (((1,), (1,)), ((), ())),
                           preferred_element_type=jnp.float32)


_BITS = [int(np.uint32(1 << b).astype(np.int64)) - (1 << 32 if b == 31 else 0)
         for b in range(32)]


def _maxk(y):
    i = lax.bitcast_convert_type(y, jnp.int32)
    ikey = jnp.where(i < 0, i ^ jnp.int32(0x7FFFFFFF), i)
    minint = jnp.int32(-(1 << 31))
    tu = jnp.zeros(y.shape[:-1] + (1,), jnp.int32)
    for b in range(31, -1, -1):
        cand = tu | jnp.int32(_BITS[b])
        cnt = jnp.sum(jnp.where(ikey >= (cand ^ minint), 1.0, 0.0),
                      axis=-1, keepdims=True)
        tu = jnp.where(cnt >= float(K), cand, tu)
    thr = tu ^ minint
    return jnp.where(ikey >= thr, y, jnp.zeros_like(y))


def _scale_from_deg(d0, d1):
    deg = jnp.maximum(d0[:, :1] + d1[:, :1], 1.0)
    return lax.rsqrt(deg)


def _tc_in_body(x_ref, win_ref, bin_ref, w0_ref, b0_ref, d0_ref, d1_ref, o_ref):
    h = jnp.maximum(_dotT(x_ref[...], win_ref[...]) + bin_ref[...], 0.0)
    y = _dotT(h, w0_ref[...]) + b0_ref[...]
    o_ref[...] = _maxk(y) * _scale_from_deg(d0_ref[...], d1_ref[...])


def _tc_mid_body(s0_ref, s1_ref, d0_ref, d1_ref, cw_ref, cb_ref,
                 lw_ref, lb_ref, o_ref):
    a = _scale_from_deg(d0_ref[...], d1_ref[...])
    h = _dotT((s0_ref[...] + s1_ref[...]) * a, cw_ref[...]) + cb_ref[...]
    y = _dotT(h, lw_ref[...]) + lb_ref[...]
    o_ref[...] = _maxk(y) * a


def _tc_out_body(s0_ref, s1_ref, d0_ref, d1_ref, cw_ref, cb_ref,
                 wo_ref, bo_ref, o_ref):
    a = _scale_from_deg(d0_ref[...], d1_ref[...])
    h = _dotT((s0_ref[...] + s1_ref[...]) * a, cw_ref[...]) + cb_ref[...]
    o_ref[...] = _dotT(h, wo_ref[...]) + bo_ref[...]


def _row_spec(cols):
    return pl.BlockSpec((BR, cols), lambda i: (i, 0))


def _full_spec(rows, cols):
    return pl.BlockSpec((rows, cols), lambda i: (0, 0))


def _tc_call(body, in_specs, out_rows):
    return pl.pallas_call(
        body,
        grid=(GRID,),
        in_specs=in_specs,
        out_specs=_row_spec(D),
        out_shape=jax.ShapeDtypeStruct((out_rows, D), jnp.float32),
    )


def kernel(x, edge_index, W_in, b_in, lin_W, lin_b, conv_W, conv_b,
           W_out, b_out):
    src = edge_index[0].astype(jnp.int32)
    dst = edge_index[1].astype(jnp.int32)
    pad = E_PAD - E
    src_p = jnp.concatenate([src, jnp.zeros((pad,), jnp.int32)])
    dst_p = jnp.concatenate([dst, jnp.full((pad,), N_PAD - 1, jnp.int32)])
    src2d = src_p.reshape(NW * CPW, CH)
    dst2d = dst_p.reshape(NW * CPW, CH)

    onesD = jnp.ones((CH, D), jnp.float32)
    zerosD = jnp.zeros((N_PAD, D), jnp.float32)

    degp = _sc_deg(dst2d, zerosD, onesD)
    d0 = degp[:N_PAD]
    d1 = degp[N_PAD:]

    deg_specs = [_row_spec(D), _row_spec(D)]
    w_spec = _full_spec(D, D)
    b_spec = _full_spec(1, D)

    hp = _tc_call(
        _tc_in_body,
        [_row_spec(D), w_spec, b_spec, w_spec, b_spec] + deg_specs,
        N,
    )(x, W_in, b_in.reshape(1, D), lin_W[0], lin_b[0].reshape(1, D), d0, d1)

    for i in range(L):
        sp = _sc_spmm(hp, src2d, dst2d, zerosD)
        s0 = sp[:N_PAD]
        s1 = sp[N_PAD:]
        if i < L - 1:
            hp = _tc_call(
                _tc_mid_body,
                [_row_spec(D), _row_spec(D)] + deg_specs
                + [w_spec, b_spec, w_spec, b_spec],
                N,
            )(s0, s1, d0, d1, conv_W[i], conv_b[i].reshape(1, D),
              lin_W[i + 1], lin_b[i + 1].reshape(1, D))
        else:
            out = _tc_call(
                _tc_out_body,
                [_row_spec(D), _row_spec(D)] + deg_specs
                + [w_spec, b_spec, w_spec, b_spec],
                N,
            )(s0, s1, d0, d1, conv_W[i], conv_b[i].reshape(1, D),
              W_out, b_out.reshape(1, D))
    return out


# speedup vs baseline: 1.2136x; 1.0006x over previous
import functools

import jax
import jax.numpy as jnp
import numpy as np
from jax import lax
from jax.experimental import pallas as pl
from jax.experimental.pallas import tpu as pltpu
from jax.experimental.pallas import tpu_sc as plsc

N = 10000
E = 320000
D = 128
L = 3
K = 32

NC = 2
NS = 16
NW = NC * NS

CH = 128
CPW = 8 * -(-E // (CH * NW * 8))
E_PAD = CPW * CH * NW
N_PAD = 10112
RPS = N_PAD // NS

BR = 512
GRID = -(-N // BR)

_SC_MESH = dict(
    mesh=plsc.VectorSubcoreMesh(core_axis_name="c", subcore_axis_name="s"),
)

@functools.partial(
    pl.kernel,
    out_type=jax.ShapeDtypeStruct((NC * N_PAD, D), jnp.float32),
    scratch_types=[
        pltpu.VMEM((CPW, CH), jnp.int32),
        pltpu.VMEM((CH, D), jnp.float32),
        pltpu.VMEM_SHARED((N_PAD, D), jnp.float32),
    ],
    **_SC_MESH,
)
def _sc_deg(dst_hbm, zeros_hbm, ones_hbm, out_hbm, dst_v, ones_v, acc_sh):
    c = lax.axis_index("c")
    s = lax.axis_index("s")
    wid = c * NS + s
    r0 = s * RPS
    pltpu.sync_copy(zeros_hbm.at[pl.ds(r0, RPS)], acc_sh.at[pl.ds(r0, RPS)])
    pltpu.sync_copy(dst_hbm.at[pl.ds(wid * CPW, CPW)], dst_v)
    pltpu.sync_copy(ones_hbm, ones_v)
    plsc.subcore_barrier()

    def body(j, carry):
        pltpu.sync_copy(ones_v, acc_sh.at[dst_v.at[j]], add=True)
        return carry

    lax.fori_loop(0, CPW, body, 0)
    plsc.subcore_barrier()
    pltpu.sync_copy(acc_sh.at[pl.ds(r0, RPS)],
                    out_hbm.at[pl.ds(c * N_PAD + r0, RPS)])


@functools.partial(
    pl.kernel,
    out_type=jax.ShapeDtypeStruct((NC * N_PAD, D), jnp.float32),
    scratch_types=[
        pltpu.VMEM((CPW // 2, CH), jnp.int32),
        pltpu.VMEM((CPW // 2, CH), jnp.int32),
        pltpu.VMEM((CH, D), jnp.float32),
        pltpu.VMEM((CH, D), jnp.float32),
        pltpu.VMEM_SHARED((N_PAD, D), jnp.float32),
        pltpu.SemaphoreType.DMA,
        pltpu.SemaphoreType.DMA,
        pltpu.SemaphoreType.DMA,
        pltpu.SemaphoreType.DMA,
    ],
    **_SC_MESH,
)
def _sc_spmm(hp_hbm, src_hbm, dst_hbm, zeros_hbm, out_hbm,
             src_v, dst_v, rows0, rows1, acc_sh, gs0, gs1, ss0, ss1):
    c = lax.axis_index("c")
    s = lax.axis_index("s")
    wid = c * NS + s
    r0 = s * RPS
    HP = CPW // 2
    pltpu.sync_copy(zeros_hbm.at[pl.ds(r0, RPS)], acc_sh.at[pl.ds(r0, RPS)])
    plsc.subcore_barrier()

    bufs = (rows0, rows1)
    gsems = (gs0, gs1)
    ssems = (ss0, ss1)

    for p in range(2):
        pltpu.sync_copy(src_hbm.at[pl.ds(wid * CPW + p * HP, HP)], src_v)
        pltpu.sync_copy(dst_hbm.at[pl.ds(wid * CPW + p * HP, HP)], dst_v)
        pltpu.async_copy(hp_hbm.at[src_v.at[0]], rows0, gs0)
        pltpu.async_copy(hp_hbm.at[src_v.at[1]], rows1, gs1)

        def body(jj, carry):
            for b in range(2):
                j = jj * 2 + b
                pltpu.make_async_copy(hp_hbm.at[src_v.at[j]], bufs[b],
                                      gsems[b]).wait()
                pltpu.async_copy(bufs[b], acc_sh.at[dst_v.at[j]], ssems[b],
                                 add=True)
                nxt = j + 2

                @pl.when(nxt < HP)
                def _():
                    pltpu.make_async_copy(bufs[b], acc_sh.at[dst_v.at[j]],
                                          ssems[b]).wait()
                    pltpu.async_copy(hp_hbm.at[src_v.at[nxt]], bufs[b],
                                     gsems[b])

            return carry

        lax.fori_loop(0, HP // 2, body, 0)
        for b in range(2):
            pltpu.make_async_copy(bufs[b], acc_sh.at[dst_v.at[HP - 2 + b]],
                                  ssems[b]).wait()
    plsc.subcore_barrier()
    pltpu.sync_copy(acc_sh.at[pl.ds(r0, RPS)],
                    out_hbm.at[pl.ds(c * N_PAD + r0, RPS)])


def _dotT(x, w):
    return lax.dot_general(x, w, (((1,), (1,)), ((), ())),
                           preferred_element_type=jnp.float32)


_BITS = [int(np.uint32(1 << b).astype(np.int64)) - (1 << 32 if b == 31 else 0)
         for b in range(32)]


def _maxk(y):
    i = lax.bitcast_convert_type(y, jnp.int32)
    ikey = jnp.where(i < 0, i ^ jnp.int32(0x7FFFFFFF), i)
    minint = jnp.int32(-(1 << 31))
    tu = jnp.zeros(y.shape[:-1] + (1,), jnp.int32)
    for b in range(31, -1, -1):
        cand = tu | jnp.int32(_BITS[b])
        cnt = jnp.sum(jnp.where(ikey >= (cand ^ minint), 1.0, 0.0),
                      axis=-1, keepdims=True)
        tu = jnp.where(cnt >= float(K), cand, tu)
    thr = tu ^ minint
    return jnp.where(ikey >= thr, y, jnp.zeros_like(y))


def _scale_from_deg(d0, d1):
    deg = jnp.maximum(d0[:, :1] + d1[:, :1], 1.0)
    return lax.rsqrt(deg)


def _tc_in_body(x_ref, win_ref, bin_ref, w0_ref, b0_ref, d0_ref, d1_ref, o_ref):
    h = jnp.maximum(_dotT(x_ref[...], win_ref[...]) + bin_ref[...], 0.0)
    y = _dotT(h, w0_ref[...]) + b0_ref[...]
    o_ref[...] = _maxk(y) * _scale_from_deg(d0_ref[...], d1_ref[...])


def _tc_mid_body(s0_ref, s1_ref, d0_ref, d1_ref, cw_ref, cb_ref,
                 lw_ref, lb_ref, o_ref):
    a = _scale_from_deg(d0_ref[...], d1_ref[...])
    h = _dotT((s0_ref[...] + s1_ref[...]) * a, cw_ref[...]) + cb_ref[...]
    y = _dotT(h, lw_ref[...]) + lb_ref[...]
    o_ref[...] = _maxk(y) * a


def _tc_out_body(s0_ref, s1_ref, d0_ref, d1_ref, cw_ref, cb_ref,
                 wo_ref, bo_ref, o_ref):
    a = _scale_from_deg(d0_ref[...], d1_ref[...])
    h = _dotT((s0_ref[...] + s1_ref[...]) * a, cw_ref[...]) + cb_ref[...]
    o_ref[...] = _dotT(h, wo_ref[...]) + bo_ref[...]


def _row_spec(cols):
    return pl.BlockSpec((BR, cols), lambda i: (i, 0))


def _full_spec(rows, cols):
    return pl.BlockSpec((rows, cols), lambda i: (0, 0))


def _tc_call(body, in_specs, out_rows):
    return pl.pallas_call(
        body,
        grid=(GRID,),
        in_specs=in_specs,
        out_specs=_row_spec(D),
        out_shape=jax.ShapeDtypeStruct((out_rows, D), jnp.float32),
    )


def kernel(x, edge_index, W_in, b_in, lin_W, lin_b, conv_W, conv_b,
           W_out, b_out):
    src = edge_index[0].astype(jnp.int32)
    dst = edge_index[1].astype(jnp.int32)
    pad = E_PAD - E
    src_p = jnp.concatenate([src, jnp.zeros((pad,), jnp.int32)])
    dst_p = jnp.concatenate([dst, jnp.full((pad,), N_PAD - 1, jnp.int32)])
    src2d = src_p.reshape(NW * CPW, CH)
    dst2d = dst_p.reshape(NW * CPW, CH)

    onesD = jnp.ones((CH, D), jnp.float32)
    zerosD = jnp.zeros((N_PAD, D), jnp.float32)

    degp = _sc_deg(dst2d, zerosD, onesD)
    d0 = degp[:N_PAD]
    d1 = degp[N_PAD:]

    deg_specs = [_row_spec(D), _row_spec(D)]
    w_spec = _full_spec(D, D)
    b_spec = _full_spec(1, D)

    hp = _tc_call(
        _tc_in_body,
        [_row_spec(D), w_spec, b_spec, w_spec, b_spec] + deg_specs,
        N,
    )(x, W_in, b_in.reshape(1, D), lin_W[0], lin_b[0].reshape(1, D), d0, d1)

    for i in range(L):
        sp = _sc_spmm(hp, src2d, dst2d, zerosD)
        s0 = sp[:N_PAD]
        s1 = sp[N_PAD:]
        if i < L - 1:
            hp = _tc_call(
                _tc_mid_body,
                [_row_spec(D), _row_spec(D)] + deg_specs
                + [w_spec, b_spec, w_spec, b_spec],
                N,
            )(s0, s1, d0, d1, conv_W[i], conv_b[i].reshape(1, D),
              lin_W[i + 1], lin_b[i + 1].reshape(1, D))
        else:
            out = _tc_call(
                _tc_out_body,
                [_row_spec(D), _row_spec(D)] + deg_specs
                + [w_spec, b_spec, w_spec, b_spec],
                N,
            )(s0, s1, d0, d1, conv_W[i], conv_b[i].reshape(1, D),
              W_out, b_out.reshape(1, D))
    return out
